# baseline (speedup 1.0000x reference)
import functools

import jax
import jax.numpy as jnp
from jax import lax
from jax.experimental import pallas as pl
from jax.experimental.pallas import tpu as pltpu
from jax.experimental.pallas import tpu_sc as plsc

EPS = 1e-12
BATCH_BLOCK = 16

NC = 2
NS = 16
NW = NC * NS
HID = 1024

TC_BATCH = 80
SC_ROWS = (128 - TC_BATCH) * 100
RPW = SC_ROWS // NW
CHUNK = 16
NCHUNK = 10

TC_BATCH = 64
SC_ROWS = (128 - TC_BATCH) * 100
RPW = SC_ROWS // NW
CHUNK = 8
NCHUNK = RPW // CHUNK
NBUF = 4


def _ln(x, gamma, beta):
    mu = jnp.mean(x, axis=-1, keepdims=True)
    xc = x - mu
    var = jnp.mean(xc * xc, axis=-1, keepdims=True)
    return xc * jax.lax.rsqrt(var + EPS) * gamma + beta


def _tc_kernel(raw_ref, pos_ref, ag_ref, ab_ref, eg_ref, eb_ref, out_ref):
    emb = _ln(pos_ref[...], eg_ref[0], eb_ref[0])
    x = raw_ref[...]
    out_ref[...] = _ln(x, ag_ref[0], ab_ref[0]) + emb[None, :, :]


def _sc_copy(raw_hbm, out_hbm, b0, b1, b2, b3, s0, s1, s2, s3,
             t0, t1, t2, t3):
    wid = lax.axis_index("s") * NC + lax.axis_index("c")
    rbase = TC_BATCH * 100 + wid * RPW
    wbase = wid * RPW
    bufs = (b0, b1, b2, b3)
    isems = (s0, s1, s2, s3)
    osems = (t0, t1, t2, t3)

    hins = [None] * NCHUNK
    houts = [None] * NCHUNK

    def start_in(j):
        hins[j] = pltpu.async_copy(
            raw_hbm.at[pl.ds(rbase + j * CHUNK, CHUNK), :],
            bufs[j % NBUF], isems[j % NBUF])

    start_in(0)
    start_in(1)
    for i in range(NCHUNK):
        nxt = i + 2
        if nxt < NCHUNK:
            if nxt >= NBUF:
                houts[nxt - NBUF].wait()
            start_in(nxt)
        hins[i].wait()
        houts[i] = pltpu.async_copy(
            bufs[i % NBUF], out_hbm.at[pl.ds(wbase + i * CHUNK, CHUNK), :],
            osems[i % NBUF])
    for i in range(NCHUNK - NBUF + 2, NCHUNK):
        houts[i].wait()


def kernel(raw_dec_emb, pos_table, ans_gamma, ans_beta, emb_gamma, emb_beta):
    batch, seq, hidden = raw_dec_emb.shape

    tc_out = pl.pallas_call(
        _tc_kernel,
        grid=(TC_BATCH // BATCH_BLOCK,),
        in_specs=[
            pl.BlockSpec((BATCH_BLOCK, seq, hidden), lambda i: (i, 0, 0)),
            pl.BlockSpec((seq, hidden), lambda i: (0, 0)),
            pl.BlockSpec((1, hidden), lambda i: (0, 0)),
            pl.BlockSpec((1, hidden), lambda i: (0, 0)),
            pl.BlockSpec((1, hidden), lambda i: (0, 0)),
            pl.BlockSpec((1, hidden), lambda i: (0, 0)),
        ],
        out_specs=pl.BlockSpec((BATCH_BLOCK, seq, hidden), lambda i: (i, 0, 0)),
        out_shape=jax.ShapeDtypeStruct((TC_BATCH, seq, hidden), jnp.float32),
        compiler_params=pltpu.CompilerParams(
            dimension_semantics=("arbitrary",),
        ),
    )(raw_dec_emb, pos_table,
      ans_gamma.reshape(1, hidden), ans_beta.reshape(1, hidden),
      emb_gamma.reshape(1, hidden), emb_beta.reshape(1, hidden))

    flat_hi = raw_dec_emb.reshape(batch * seq, HID)
    mesh = plsc.VectorSubcoreMesh(core_axis_name="c", subcore_axis_name="s")
    sc_out = functools.partial(
        pl.kernel,
        out_type=jax.ShapeDtypeStruct((SC_ROWS, HID), jnp.float32),
        mesh=mesh,
        scratch_types=[
            pltpu.VMEM((CHUNK, HID), jnp.float32),
            pltpu.VMEM((CHUNK, HID), jnp.float32),
            pltpu.VMEM((CHUNK, HID), jnp.float32),
            pltpu.VMEM((CHUNK, HID), jnp.float32),
            pltpu.SemaphoreType.DMA,
            pltpu.SemaphoreType.DMA,
            pltpu.SemaphoreType.DMA,
            pltpu.SemaphoreType.DMA,
            pltpu.SemaphoreType.DMA,
            pltpu.SemaphoreType.DMA,
            pltpu.SemaphoreType.DMA,
            pltpu.SemaphoreType.DMA,
        ],
    )(_sc_copy)(flat_hi)

    return jnp.concatenate(
        [tc_out, sc_out.reshape(batch - TC_BATCH, seq, hidden)], axis=0)


# speedup vs baseline: 1.7819x; 1.7819x over previous
import jax
import jax.numpy as jnp
from jax.experimental import pallas as pl
from jax.experimental.pallas import tpu as pltpu

EPS = 1e-12
BATCH_BLOCK = 16


def _ln(x, gamma, beta):
    mu = jnp.mean(x, axis=-1, keepdims=True)
    musq = jnp.mean(x * x, axis=-1, keepdims=True)
    var = musq - mu * mu
    rs = jax.lax.rsqrt(var + EPS)
    return (x - mu) * rs * gamma + beta


def _fused_kernel(raw_ref, pos_ref, ag_ref, ab_ref, eg_ref, eb_ref,
                  out_ref, emb_ref):
    @pl.when(pl.program_id(0) == 0)
    def _():
        emb_ref[...] = _ln(pos_ref[...], eg_ref[0], eb_ref[0])

    x = raw_ref[...]
    out_ref[...] = _ln(x, ag_ref[0], ab_ref[0]) + emb_ref[...][None, :, :]


def kernel(raw_dec_emb, pos_table, ans_gamma, ans_beta, emb_gamma, emb_beta):
    batch, seq, hidden = raw_dec_emb.shape
    grid = batch // BATCH_BLOCK
    return pl.pallas_call(
        _fused_kernel,
        grid=(grid,),
        in_specs=[
            pl.BlockSpec((BATCH_BLOCK, seq, hidden), lambda i: (i, 0, 0)),
            pl.BlockSpec((seq, hidden), lambda i: (0, 0)),
            pl.BlockSpec((1, hidden), lambda i: (0, 0)),
            pl.BlockSpec((1, hidden), lambda i: (0, 0)),
            pl.BlockSpec((1, hidden), lambda i: (0, 0)),
            pl.BlockSpec((1, hidden), lambda i: (0, 0)),
        ],
        out_specs=pl.BlockSpec((BATCH_BLOCK, seq, hidden), lambda i: (i, 0, 0)),
        out_shape=jax.ShapeDtypeStruct((batch, seq, hidden), raw_dec_emb.dtype),
        scratch_shapes=[pltpu.VMEM((seq, hidden), jnp.float32)],
        compiler_params=pltpu.CompilerParams(
            dimension_semantics=("arbitrary",),
        ),
    )(raw_dec_emb, pos_table,
      ans_gamma.reshape(1, hidden), ans_beta.reshape(1, hidden),
      emb_gamma.reshape(1, hidden), emb_beta.reshape(1, hidden))


# speedup vs baseline: 1.7899x; 1.0045x over previous
import functools

import jax
import jax.numpy as jnp
from jax.experimental import pallas as pl
from jax.experimental.pallas import tpu as pltpu

EPS = 1e-12
BATCH_BLOCK = 16


def _ln(x, gamma, beta):
    mu = jnp.mean(x, axis=-1, keepdims=True)
    xc = x - mu
    var = jnp.mean(xc * xc, axis=-1, keepdims=True)
    return xc * jax.lax.rsqrt(var + EPS) * gamma + beta


def _fused_kernel(raw_ref, pos_ref, ag_ref, ab_ref, eg_ref, eb_ref,
                  out_ref, emb_ref):
    @pl.when(pl.program_id(0) == 0)
    def _():
        emb_ref[...] = _ln(pos_ref[...], eg_ref[0], eb_ref[0])

    x = raw_ref[...]
    out_ref[...] = _ln(x, ag_ref[0], ab_ref[0]) + emb_ref[...][None, :, :]


def kernel(raw_dec_emb, pos_table, ans_gamma, ans_beta, emb_gamma, emb_beta):
    batch, seq, hidden = raw_dec_emb.shape
    grid = batch // BATCH_BLOCK
    return pl.pallas_call(
        _fused_kernel,
        grid=(grid,),
        in_specs=[
            pl.BlockSpec((BATCH_BLOCK, seq, hidden), lambda i: (i, 0, 0)),
            pl.BlockSpec((seq, hidden), lambda i: (0, 0)),
            pl.BlockSpec((1, hidden), lambda i: (0, 0)),
            pl.BlockSpec((1, hidden), lambda i: (0, 0)),
            pl.BlockSpec((1, hidden), lambda i: (0, 0)),
            pl.BlockSpec((1, hidden), lambda i: (0, 0)),
        ],
        out_specs=pl.BlockSpec((BATCH_BLOCK, seq, hidden), lambda i: (i, 0, 0)),
        out_shape=jax.ShapeDtypeStruct((batch, seq, hidden), raw_dec_emb.dtype),
        scratch_shapes=[pltpu.VMEM((seq, hidden), jnp.float32)],
        compiler_params=pltpu.CompilerParams(
            dimension_semantics=("arbitrary",),
        ),
    )(raw_dec_emb, pos_table,
      ans_gamma.reshape(1, hidden), ans_beta.reshape(1, hidden),
      emb_gamma.reshape(1, hidden), emb_beta.reshape(1, hidden))
